# baseline (speedup 1.0000x reference)
import functools

import jax
import jax.numpy as jnp
from jax import lax
from jax.experimental import pallas as pl
from jax.experimental.pallas import tpu as pltpu
from jax.experimental.pallas import tpu_sc as plsc

NC = 2
NS = 16
NW = NC * NS
CH = 128


def _make_agg(n, n_acc, d, epw):
    n_chunks = epw // CH
    rz = n_acc // NS
    last = n - (NS - 1) * rz
    assert 0 < last <= rz and last % 8 == 0 and rz % 8 == 0
    mesh = plsc.VectorSubcoreMesh(core_axis_name="c", subcore_axis_name="s")

    assert n_chunks % 2 == 0

    @functools.partial(
        pl.kernel,
        out_type=jax.ShapeDtypeStruct((NC, n, d), jnp.float32),
        mesh=mesh,
        scratch_types=[
            pltpu.VMEM((CH,), jnp.int32),
            pltpu.VMEM((CH,), jnp.int32),
            pltpu.VMEM((CH,), jnp.int32),
            pltpu.VMEM((CH,), jnp.int32),
            pltpu.VMEM((CH, d), jnp.float32),
            pltpu.VMEM((CH, d), jnp.float32),
            pltpu.VMEM_SHARED((n_acc, d), jnp.float32),
        ] + [pltpu.SemaphoreType.DMA] * 8,
    )
    def agg(feat_hbm, src_hbm, dst_hbm, zeros_hbm, out_hbm,
            sidx0, sidx1, didx0, didx1, rows0, rows1, acc_sh,
            i0, i1, d0, d1, g0, g1, s0, s1):
        c = lax.axis_index("c")
        s = lax.axis_index("s")
        wid = s * NC + c
        base = wid * epw

        sidx = (sidx0, sidx1)
        didx = (didx0, didx1)
        rows = (rows0, rows1)
        isem = (i0, i1)
        dsem = (d0, d1)
        gsem = (g0, g1)
        ssem = (s0, s1)

        def si_copy(k, b):
            off = base + k * CH
            return pltpu.make_async_copy(src_hbm.at[pl.ds(off, CH)],
                                         sidx[b], isem[b])

        def di_copy(k, b):
            off = base + k * CH
            return pltpu.make_async_copy(dst_hbm.at[pl.ds(off, CH)],
                                         didx[b], dsem[b])

        def g_copy(b):
            return pltpu.make_async_copy(feat_hbm.at[sidx[b]], rows[b],
                                         gsem[b])

        def s_copy(b):
            return pltpu.make_async_copy(rows[b], acc_sh.at[didx[b]], ssem[b])

        for b in (0, 1):
            si_copy(b, b).start()
            di_copy(b, b).start()
        z0 = s * rz
        pltpu.sync_copy(zeros_hbm.at[pl.ds(z0, rz)], acc_sh.at[pl.ds(z0, rz)])
        plsc.subcore_barrier()

        def body(j, carry):
            c0 = 2 * j

            @pl.when(j > 0)
            def _():
                s_copy(0).wait()
                di_copy(c0, 0).start()
            si_copy(c0, 0).wait()
            g_copy(0).start()

            @pl.when(j > 0)
            def _():
                s_copy(1).wait()
                di_copy(c0 + 1, 1).start()
            si_copy(c0 + 1, 1).wait()
            g_copy(1).start()

            g_copy(0).wait()

            @pl.when(j < n_chunks // 2 - 1)
            def _():
                si_copy(c0 + 2, 0).start()
            di_copy(c0, 0).wait()
            s_copy(0).start(add=True)

            g_copy(1).wait()

            @pl.when(j < n_chunks // 2 - 1)
            def _():
                si_copy(c0 + 3, 1).start()
            di_copy(c0 + 1, 1).wait()
            s_copy(1).start(add=True)

            return carry

        lax.fori_loop(0, n_chunks // 2, body, 0)
        s_copy(0).wait()
        s_copy(1).wait()
        plsc.subcore_barrier()

        r0 = s * rz

        @pl.when(s < NS - 1)
        def _():
            pltpu.sync_copy(acc_sh.at[pl.ds(r0, rz)],
                            out_hbm.at[c, pl.ds(r0, rz)])

        @pl.when(s == NS - 1)
        def _():
            pltpu.sync_copy(acc_sh.at[pl.ds((NS - 1) * rz, last)],
                            out_hbm.at[c, pl.ds((NS - 1) * rz, last)])

    return agg


def _dense_body(x_ref, p0_ref, p1_ref, w_ref, b_ref, o_ref):
    h = x_ref[...] + p0_ref[...] + p1_ref[...]
    y = lax.dot_general(h, w_ref[...], (((1,), (0,)), ((), ())),
                        preferred_element_type=jnp.float32,
                        precision=lax.Precision.HIGHEST)
    o_ref[...] = jnp.maximum(y + b_ref[...], 0.0)


def _make_dense(n, d, h):
    return pl.pallas_call(
        _dense_body,
        out_shape=jax.ShapeDtypeStruct((n, h), jnp.float32),
    )


def _make_final(n, d, h, g, cls):
    def body(h1_ref, p0_ref, p1_ref, w2_ref, b2_ref, gid_ref, wo_ref, bo_ref,
             o_ref):
        x = h1_ref[...] + p0_ref[...] + p1_ref[...]
        y = lax.dot_general(x, w2_ref[...], (((1,), (0,)), ((), ())),
                            preferred_element_type=jnp.float32,
                            precision=lax.Precision.HIGHEST)
        h2 = jnp.maximum(y + b2_ref[...], 0.0)
        gid = gid_ref[...]
        onehot = (gid == lax.broadcasted_iota(jnp.int32, (n, g), 1)
                  ).astype(jnp.float32)
        sums = lax.dot_general(onehot, h2, (((0,), (0,)), ((), ())),
                               preferred_element_type=jnp.float32,
                               precision=lax.Precision.HIGHEST)
        ones = jnp.ones((n, 1), jnp.float32)
        counts = lax.dot_general(onehot, ones, (((0,), (0,)), ((), ())),
                                 preferred_element_type=jnp.float32,
                                 precision=lax.Precision.HIGHEST)
        pooled = sums / jnp.maximum(counts, 1.0)
        logits = lax.dot_general(pooled, wo_ref[...], (((1,), (0,)), ((), ())),
                                 preferred_element_type=jnp.float32,
                                 precision=lax.Precision.HIGHEST) + bo_ref[...]
        m = jnp.max(logits, axis=1, keepdims=True)
        e = jnp.exp(logits - m)
        o_ref[...] = e / jnp.sum(e, axis=1, keepdims=True)

    return pl.pallas_call(
        body,
        out_shape=jax.ShapeDtypeStruct((g, cls), jnp.float32),
    )


def kernel(x, edge_index, i, W1, b1, W2, b2, Wo, bo):
    n, d = x.shape
    hid = W1.shape[1]
    g = 64
    cls = Wo.shape[1]
    e = edge_index.shape[1]

    n_acc = NS * 8 * (-(-(n + 1) // (NS * 8)))
    epw = 2 * CH * (-(-e // (NW * 2 * CH)))
    e_pad = NW * epw

    src = edge_index[0].astype(jnp.int32)
    dst = edge_index[1].astype(jnp.int32)
    pad = e_pad - e
    if pad:
        src = jnp.concatenate([src, jnp.zeros((pad,), jnp.int32)])
        dst = jnp.concatenate([dst, jnp.full((pad,), n, jnp.int32)])
    zeros = jnp.zeros((n_acc, d), jnp.float32)

    agg = _make_agg(n, n_acc, d, epw)
    dense1 = _make_dense(n, d, hid)
    final = _make_final(n, hid, hid, g, cls)

    p = agg(x, src, dst, zeros)
    h1 = dense1(x, p[0], p[1], W1, b1.reshape(1, -1))
    q = agg(h1, src, dst, zeros)
    return final(h1, q[0], q[1], W2, b2.reshape(1, -1),
                 i.astype(jnp.int32).reshape(-1, 1), Wo, bo.reshape(1, -1))


# speedup vs baseline: 1.2195x; 1.2195x over previous
import functools

import jax
import jax.numpy as jnp
from jax import lax
from jax.experimental import pallas as pl
from jax.experimental.pallas import tpu as pltpu
from jax.experimental.pallas import tpu_sc as plsc

NC = 1
NS = 16
NW = NC * NS
CH = 128


def _make_agg(n, n_acc, d, epw):
    n_chunks = epw // CH
    rz = n_acc // NS
    last = n - (NS - 1) * rz
    assert 0 < last <= rz and last % 8 == 0 and rz % 8 == 0
    mesh = plsc.VectorSubcoreMesh(core_axis_name="c", subcore_axis_name="s",
                                  num_cores=NC)

    assert n_chunks % 2 == 0

    @functools.partial(
        pl.kernel,
        out_type=jax.ShapeDtypeStruct((NC, n, d), jnp.float32),
        mesh=mesh,
        scratch_types=[
            pltpu.VMEM((CH,), jnp.int32),
            pltpu.VMEM((CH,), jnp.int32),
            pltpu.VMEM((CH,), jnp.int32),
            pltpu.VMEM((CH,), jnp.int32),
            pltpu.VMEM((CH, d), jnp.float32),
            pltpu.VMEM((CH, d), jnp.float32),
            pltpu.VMEM_SHARED((n_acc, d), jnp.float32),
        ] + [pltpu.SemaphoreType.DMA] * 8,
    )
    def agg(feat_hbm, src_hbm, dst_hbm, zeros_hbm, out_hbm,
            sidx0, sidx1, didx0, didx1, rows0, rows1, acc_sh,
            i0, i1, d0, d1, g0, g1, s0, s1):
        c = lax.axis_index("c")
        s = lax.axis_index("s")
        wid = s * NC + c
        base = wid * epw

        sidx = (sidx0, sidx1)
        didx = (didx0, didx1)
        rows = (rows0, rows1)
        isem = (i0, i1)
        dsem = (d0, d1)
        gsem = (g0, g1)
        ssem = (s0, s1)

        def si_copy(k, b):
            off = base + k * CH
            return pltpu.make_async_copy(src_hbm.at[pl.ds(off, CH)],
                                         sidx[b], isem[b])

        def di_copy(k, b):
            off = base + k * CH
            return pltpu.make_async_copy(dst_hbm.at[pl.ds(off, CH)],
                                         didx[b], dsem[b])

        def g_copy(b):
            return pltpu.make_async_copy(feat_hbm.at[sidx[b]], rows[b],
                                         gsem[b])

        def s_copy(b):
            return pltpu.make_async_copy(rows[b], acc_sh.at[didx[b]], ssem[b])

        for b in (0, 1):
            si_copy(b, b).start()
            di_copy(b, b).start()
        z0 = s * rz
        pltpu.sync_copy(zeros_hbm.at[pl.ds(z0, rz)], acc_sh.at[pl.ds(z0, rz)])
        plsc.subcore_barrier()

        def body(j, carry):
            c0 = 2 * j

            @pl.when(j > 0)
            def _():
                s_copy(0).wait()
                di_copy(c0, 0).start()
            si_copy(c0, 0).wait()
            g_copy(0).start()

            @pl.when(j > 0)
            def _():
                s_copy(1).wait()
                di_copy(c0 + 1, 1).start()
            si_copy(c0 + 1, 1).wait()
            g_copy(1).start()

            g_copy(0).wait()

            @pl.when(j < n_chunks // 2 - 1)
            def _():
                si_copy(c0 + 2, 0).start()
            di_copy(c0, 0).wait()
            s_copy(0).start(add=True)

            g_copy(1).wait()

            @pl.when(j < n_chunks // 2 - 1)
            def _():
                si_copy(c0 + 3, 1).start()
            di_copy(c0 + 1, 1).wait()
            s_copy(1).start(add=True)

            return carry

        lax.fori_loop(0, n_chunks // 2, body, 0)
        s_copy(0).wait()
        s_copy(1).wait()
        plsc.subcore_barrier()

        r0 = s * rz

        @pl.when(s < NS - 1)
        def _():
            pltpu.sync_copy(acc_sh.at[pl.ds(r0, rz)],
                            out_hbm.at[c, pl.ds(r0, rz)])

        @pl.when(s == NS - 1)
        def _():
            pltpu.sync_copy(acc_sh.at[pl.ds((NS - 1) * rz, last)],
                            out_hbm.at[c, pl.ds((NS - 1) * rz, last)])

    return agg


def _dense_body(x_ref, p_ref, w_ref, b_ref, o_ref):
    h = x_ref[...]
    for k in range(p_ref.shape[0]):
        h = h + p_ref[k]
    y = lax.dot_general(h, w_ref[...], (((1,), (0,)), ((), ())),
                        preferred_element_type=jnp.float32,
                        precision=lax.Precision.HIGHEST)
    o_ref[...] = jnp.maximum(y + b_ref[...], 0.0)


def _make_dense(n, d, h):
    return pl.pallas_call(
        _dense_body,
        out_shape=jax.ShapeDtypeStruct((n, h), jnp.float32),
    )


def _make_final(n, d, h, g, cls):
    def body(h1_ref, p_ref, w2_ref, b2_ref, gid_ref, wo_ref, bo_ref,
             o_ref):
        x = h1_ref[...]
        for k in range(p_ref.shape[0]):
            x = x + p_ref[k]
        y = lax.dot_general(x, w2_ref[...], (((1,), (0,)), ((), ())),
                            preferred_element_type=jnp.float32,
                            precision=lax.Precision.HIGHEST)
        h2 = jnp.maximum(y + b2_ref[...], 0.0)
        gid = gid_ref[...]
        onehot = (gid == lax.broadcasted_iota(jnp.int32, (n, g), 1)
                  ).astype(jnp.float32)
        sums = lax.dot_general(onehot, h2, (((0,), (0,)), ((), ())),
                               preferred_element_type=jnp.float32,
                               precision=lax.Precision.HIGHEST)
        ones = jnp.ones((n, 1), jnp.float32)
        counts = lax.dot_general(onehot, ones, (((0,), (0,)), ((), ())),
                                 preferred_element_type=jnp.float32,
                                 precision=lax.Precision.HIGHEST)
        pooled = sums / jnp.maximum(counts, 1.0)
        logits = lax.dot_general(pooled, wo_ref[...], (((1,), (0,)), ((), ())),
                                 preferred_element_type=jnp.float32,
                                 precision=lax.Precision.HIGHEST) + bo_ref[...]
        m = jnp.max(logits, axis=1, keepdims=True)
        e = jnp.exp(logits - m)
        o_ref[...] = e / jnp.sum(e, axis=1, keepdims=True)

    return pl.pallas_call(
        body,
        out_shape=jax.ShapeDtypeStruct((g, cls), jnp.float32),
    )


def kernel(x, edge_index, i, W1, b1, W2, b2, Wo, bo):
    n, d = x.shape
    hid = W1.shape[1]
    g = 64
    cls = Wo.shape[1]
    e = edge_index.shape[1]

    n_acc = NS * 8 * (-(-(n + 1) // (NS * 8)))
    epw = 2 * CH * (-(-e // (NW * 2 * CH)))
    e_pad = NW * epw

    src = edge_index[0].astype(jnp.int32)
    dst = edge_index[1].astype(jnp.int32)
    pad = e_pad - e
    if pad:
        src = jnp.concatenate([src, jnp.zeros((pad,), jnp.int32)])
        dst = jnp.concatenate([dst, jnp.full((pad,), n, jnp.int32)])
    zeros = jnp.zeros((n_acc, d), jnp.float32)

    agg = _make_agg(n, n_acc, d, epw)
    dense1 = _make_dense(n, d, hid)
    final = _make_final(n, hid, hid, g, cls)

    p = agg(x, src, dst, zeros)
    h1 = dense1(x, p, W1, b1.reshape(1, -1))
    q = agg(h1, src, dst, zeros)
    return final(h1, q, W2, b2.reshape(1, -1),
                 i.astype(jnp.int32).reshape(-1, 1), Wo, bo.reshape(1, -1))


# speedup vs baseline: 1.7282x; 1.4171x over previous
import functools

import jax
import jax.numpy as jnp
from jax import lax
from jax.experimental import pallas as pl
from jax.experimental.pallas import tpu as pltpu
from jax.experimental.pallas import tpu_sc as plsc

NC = 2
NS = 16
CH = 128


def _make_agg(n, n_acc, dh, epw):
    n_chunks = epw // CH
    rz = n_acc // NS
    last = n - (NS - 1) * rz
    assert 0 < last <= rz and last % 8 == 0 and rz % 8 == 0
    assert n_chunks % 2 == 0
    mesh = plsc.VectorSubcoreMesh(core_axis_name="c", subcore_axis_name="s",
                                  num_cores=NC)

    @functools.partial(
        pl.kernel,
        out_type=jax.ShapeDtypeStruct((NC, n, dh), jnp.float32),
        mesh=mesh,
        scratch_types=[
            pltpu.VMEM((CH,), jnp.int32),
            pltpu.VMEM((CH,), jnp.int32),
            pltpu.VMEM((CH,), jnp.int32),
            pltpu.VMEM((CH,), jnp.int32),
            pltpu.VMEM((CH, dh), jnp.float32),
            pltpu.VMEM((CH, dh), jnp.float32),
            pltpu.VMEM_SHARED((n_acc, dh), jnp.float32),
        ] + [pltpu.SemaphoreType.DMA] * 8,
        compiler_params=pltpu.CompilerParams(use_tc_tiling_on_sc=False),
    )
    def agg(feat2, src2_hbm, dst_hbm, zeros_hbm, out_hbm,
            sidx0, sidx1, didx0, didx1, rows0, rows1, acc_sh,
            i0, i1, d0, d1, g0, g1, s0, s1):
        c = lax.axis_index("c")
        s = lax.axis_index("s")
        base = s * epw

        sidx = (sidx0, sidx1)
        didx = (didx0, didx1)
        rows = (rows0, rows1)
        isem = (i0, i1)
        dsem = (d0, d1)
        gsem = (g0, g1)
        ssem = (s0, s1)

        def si_copy(k, b):
            off = base + k * CH
            return pltpu.make_async_copy(src2_hbm.at[c, pl.ds(off, CH)],
                                         sidx[b], isem[b])

        def di_copy(k, b):
            off = base + k * CH
            return pltpu.make_async_copy(dst_hbm.at[pl.ds(off, CH)],
                                         didx[b], dsem[b])

        def g_copy(b):
            return pltpu.make_async_copy(feat2.at[sidx[b]], rows[b],
                                         gsem[b])

        def s_copy(b):
            return pltpu.make_async_copy(rows[b], acc_sh.at[didx[b]], ssem[b])

        for b in (0, 1):
            si_copy(b, b).start()
            di_copy(b, b).start()
        z0 = s * rz
        pltpu.sync_copy(zeros_hbm.at[pl.ds(z0, rz)], acc_sh.at[pl.ds(z0, rz)])
        plsc.subcore_barrier()

        def body(j, carry):
            c0 = 2 * j

            @pl.when(j > 0)
            def _():
                s_copy(0).wait()
                di_copy(c0, 0).start()
            si_copy(c0, 0).wait()
            g_copy(0).start()

            @pl.when(j > 0)
            def _():
                s_copy(1).wait()
                di_copy(c0 + 1, 1).start()
            si_copy(c0 + 1, 1).wait()
            g_copy(1).start()

            g_copy(0).wait()

            @pl.when(j < n_chunks // 2 - 1)
            def _():
                si_copy(c0 + 2, 0).start()
            di_copy(c0, 0).wait()
            s_copy(0).start(add=True)

            g_copy(1).wait()

            @pl.when(j < n_chunks // 2 - 1)
            def _():
                si_copy(c0 + 3, 1).start()
            di_copy(c0 + 1, 1).wait()
            s_copy(1).start(add=True)

            return carry

        lax.fori_loop(0, n_chunks // 2, body, 0)
        s_copy(0).wait()
        s_copy(1).wait()
        plsc.subcore_barrier()

        r0 = s * rz

        @pl.when(s < NS - 1)
        def _():
            pltpu.sync_copy(acc_sh.at[pl.ds(r0, rz)],
                            out_hbm.at[c, pl.ds(r0, rz)])

        @pl.when(s == NS - 1)
        def _():
            pltpu.sync_copy(acc_sh.at[pl.ds((NS - 1) * rz, last)],
                            out_hbm.at[c, pl.ds((NS - 1) * rz, last)])

    return agg


def _make_dense(n, d, h):
    def body(x_ref, p_ref, w_ref, b_ref, o_ref):
        x = x_ref[...] + jnp.concatenate([p_ref[0], p_ref[1]], axis=1)
        y = lax.dot_general(x, w_ref[...], (((1,), (0,)), ((), ())),
                            preferred_element_type=jnp.float32,
                            precision=lax.Precision.HIGHEST)
        r = jnp.maximum(y + b_ref[...], 0.0)
        o_ref[pl.ds(0, n), :] = r[:, :h // 2]
        o_ref[pl.ds(n, n), :] = r[:, h // 2:]

    return pl.pallas_call(
        body,
        out_shape=jax.ShapeDtypeStruct((2 * n, h // 2), jnp.float32),
    )


def _make_final(n, h, g, cls):
    def body(h2s_ref, q_ref, w2_ref, b2_ref, gid_ref, wo_ref, bo_ref,
             o_ref):
        x = (jnp.concatenate([h2s_ref[pl.ds(0, n), :],
                              h2s_ref[pl.ds(n, n), :]], axis=1)
             + jnp.concatenate([q_ref[0], q_ref[1]], axis=1))
        y = lax.dot_general(x, w2_ref[...], (((1,), (0,)), ((), ())),
                            preferred_element_type=jnp.float32,
                            precision=lax.Precision.HIGHEST)
        h2 = jnp.maximum(y + b2_ref[...], 0.0)
        gid = gid_ref[...]
        onehot = (gid == lax.broadcasted_iota(jnp.int32, (n, g), 1)
                  ).astype(jnp.float32)
        sums = lax.dot_general(onehot, h2, (((0,), (0,)), ((), ())),
                               preferred_element_type=jnp.float32,
                               precision=lax.Precision.HIGHEST)
        ones = jnp.ones((n, 1), jnp.float32)
        counts = lax.dot_general(onehot, ones, (((0,), (0,)), ((), ())),
                                 preferred_element_type=jnp.float32,
                                 precision=lax.Precision.HIGHEST)
        pooled = sums / jnp.maximum(counts, 1.0)
        logits = lax.dot_general(pooled, wo_ref[...], (((1,), (0,)), ((), ())),
                                 preferred_element_type=jnp.float32,
                                 precision=lax.Precision.HIGHEST) + bo_ref[...]
        m = jnp.max(logits, axis=1, keepdims=True)
        e = jnp.exp(logits - m)
        o_ref[...] = e / jnp.sum(e, axis=1, keepdims=True)

    return pl.pallas_call(
        body,
        out_shape=jax.ShapeDtypeStruct((g, cls), jnp.float32),
    )


def kernel(x, edge_index, i, W1, b1, W2, b2, Wo, bo):
    n, d = x.shape
    hid = W1.shape[1]
    g = 64
    cls = Wo.shape[1]
    e = edge_index.shape[1]
    assert d % 2 == 0 and hid % 2 == 0

    n_acc = NS * 8 * (-(-(n + 1) // (NS * 8)))
    epw = 2 * CH * (-(-e // (NS * 2 * CH)))
    e_pad = NS * epw

    src = edge_index[0].astype(jnp.int32)
    dst = edge_index[1].astype(jnp.int32)
    pad = e_pad - e
    if pad:
        src = jnp.concatenate([src, jnp.zeros((pad,), jnp.int32)])
        dst = jnp.concatenate([dst, jnp.full((pad,), n, jnp.int32)])
    src2 = jnp.stack([src, src + n])
    zeros = jnp.zeros((n_acc, d // 2), jnp.float32)

    agg = _make_agg(n, n_acc, d // 2, epw)
    dense1 = _make_dense(n, d, hid)
    final = _make_final(n, hid, g, cls)

    x2 = jnp.concatenate([x[:, :d // 2], x[:, d // 2:]], axis=0)
    p = agg(x2, src2, dst, zeros)
    h2s = dense1(x, p, W1, b1.reshape(1, -1))
    q = agg(h2s, src2, dst, zeros)
    return final(h2s, q, W2, b2.reshape(1, -1),
                 i.astype(jnp.int32).reshape(-1, 1), Wo, bo.reshape(1, -1))


# speedup vs baseline: 2.5070x; 1.4507x over previous
import functools

import jax
import jax.numpy as jnp
from jax import lax
from jax.experimental import pallas as pl
from jax.experimental.pallas import tpu as pltpu
from jax.experimental.pallas import tpu_sc as plsc

NC = 2
NS = 16
CH = 128


def _make_agg(n, n_acc, dh, epw):
    n_chunks = epw // CH
    rz = n_acc // NS
    last = n - (NS - 1) * rz
    assert 0 < last <= rz and last % 8 == 0 and rz % 8 == 0
    assert n_chunks % 2 == 0
    mesh = plsc.VectorSubcoreMesh(core_axis_name="c", subcore_axis_name="s",
                                  num_cores=NC)

    @functools.partial(
        pl.kernel,
        out_type=jax.ShapeDtypeStruct((NC, n, dh), jnp.float32),
        mesh=mesh,
        scratch_types=[
            pltpu.VMEM((CH,), jnp.int32),
            pltpu.VMEM((CH,), jnp.int32),
            pltpu.VMEM((CH,), jnp.int32),
            pltpu.VMEM((CH,), jnp.int32),
            pltpu.VMEM((CH, dh), jnp.float32),
            pltpu.VMEM((CH, dh), jnp.float32),
            pltpu.VMEM_SHARED((n, dh), jnp.float32),
            pltpu.VMEM_SHARED((n_acc, dh), jnp.float32),
        ] + [pltpu.SemaphoreType.DMA] * 8,
        compiler_params=pltpu.CompilerParams(use_tc_tiling_on_sc=False),
    )
    def agg(feat2, src_hbm, dst_hbm, zeros_hbm, out_hbm,
            sidx0, sidx1, didx0, didx1, rows0, rows1, feat_sh, acc_sh,
            i0, i1, d0, d1, g0, g1, s0, s1):
        c = lax.axis_index("c")
        s = lax.axis_index("s")
        base = s * epw

        sidx = (sidx0, sidx1)
        didx = (didx0, didx1)
        rows = (rows0, rows1)
        isem = (i0, i1)
        dsem = (d0, d1)
        gsem = (g0, g1)
        ssem = (s0, s1)

        def si_copy(k, b):
            off = base + k * CH
            return pltpu.make_async_copy(src_hbm.at[pl.ds(off, CH)],
                                         sidx[b], isem[b])

        def di_copy(k, b):
            off = base + k * CH
            return pltpu.make_async_copy(dst_hbm.at[pl.ds(off, CH)],
                                         didx[b], dsem[b])

        def g_copy(b):
            return pltpu.make_async_copy(feat_sh.at[sidx[b]], rows[b],
                                         gsem[b])

        def s_copy(b):
            return pltpu.make_async_copy(rows[b], acc_sh.at[didx[b]], ssem[b])

        for b in (0, 1):
            si_copy(b, b).start()
            di_copy(b, b).start()
        z0 = s * rz

        @pl.when(s < NS - 1)
        def _():
            pltpu.sync_copy(feat2.at[pl.ds(c * n + z0, rz)],
                            feat_sh.at[pl.ds(z0, rz)])

        @pl.when(s == NS - 1)
        def _():
            pltpu.sync_copy(feat2.at[pl.ds(c * n + (NS - 1) * rz, last)],
                            feat_sh.at[pl.ds((NS - 1) * rz, last)])

        pltpu.sync_copy(zeros_hbm.at[pl.ds(z0, rz)], acc_sh.at[pl.ds(z0, rz)])
        plsc.subcore_barrier()

        def body(j, carry):
            c0 = 2 * j

            @pl.when(j > 0)
            def _():
                s_copy(0).wait()
                di_copy(c0, 0).start()
            si_copy(c0, 0).wait()
            g_copy(0).start()

            @pl.when(j > 0)
            def _():
                s_copy(1).wait()
                di_copy(c0 + 1, 1).start()
            si_copy(c0 + 1, 1).wait()
            g_copy(1).start()

            g_copy(0).wait()

            @pl.when(j < n_chunks // 2 - 1)
            def _():
                si_copy(c0 + 2, 0).start()
            di_copy(c0, 0).wait()
            s_copy(0).start(add=True)

            g_copy(1).wait()

            @pl.when(j < n_chunks // 2 - 1)
            def _():
                si_copy(c0 + 3, 1).start()
            di_copy(c0 + 1, 1).wait()
            s_copy(1).start(add=True)

            return carry

        lax.fori_loop(0, n_chunks // 2, body, 0)
        s_copy(0).wait()
        s_copy(1).wait()
        plsc.subcore_barrier()

        r0 = s * rz

        @pl.when(s < NS - 1)
        def _():
            pltpu.sync_copy(acc_sh.at[pl.ds(r0, rz)],
                            out_hbm.at[c, pl.ds(r0, rz)])

        @pl.when(s == NS - 1)
        def _():
            pltpu.sync_copy(acc_sh.at[pl.ds((NS - 1) * rz, last)],
                            out_hbm.at[c, pl.ds((NS - 1) * rz, last)])

    return agg


def _make_dense(n, d, h):
    def body(x_ref, p_ref, w_ref, b_ref, o_ref):
        x = x_ref[...] + jnp.concatenate([p_ref[0], p_ref[1]], axis=1)
        y = lax.dot_general(x, w_ref[...], (((1,), (0,)), ((), ())),
                            preferred_element_type=jnp.float32,
                            precision=lax.Precision.HIGHEST)
        r = jnp.maximum(y + b_ref[...], 0.0)
        o_ref[pl.ds(0, n), :] = r[:, :h // 2]
        o_ref[pl.ds(n, n), :] = r[:, h // 2:]

    return pl.pallas_call(
        body,
        out_shape=jax.ShapeDtypeStruct((2 * n, h // 2), jnp.float32),
    )


def _make_final(n, h, g, cls):
    def body(h2s_ref, q_ref, w2_ref, b2_ref, gid_ref, wo_ref, bo_ref,
             o_ref):
        x = (jnp.concatenate([h2s_ref[pl.ds(0, n), :],
                              h2s_ref[pl.ds(n, n), :]], axis=1)
             + jnp.concatenate([q_ref[0], q_ref[1]], axis=1))
        y = lax.dot_general(x, w2_ref[...], (((1,), (0,)), ((), ())),
                            preferred_element_type=jnp.float32,
                            precision=lax.Precision.HIGHEST)
        h2 = jnp.maximum(y + b2_ref[...], 0.0)
        gid = gid_ref[...]
        onehot = (gid == lax.broadcasted_iota(jnp.int32, (n, g), 1)
                  ).astype(jnp.float32)
        sums = lax.dot_general(onehot, h2, (((0,), (0,)), ((), ())),
                               preferred_element_type=jnp.float32,
                               precision=lax.Precision.HIGHEST)
        ones = jnp.ones((n, 1), jnp.float32)
        counts = lax.dot_general(onehot, ones, (((0,), (0,)), ((), ())),
                                 preferred_element_type=jnp.float32,
                                 precision=lax.Precision.HIGHEST)
        pooled = sums / jnp.maximum(counts, 1.0)
        logits = lax.dot_general(pooled, wo_ref[...], (((1,), (0,)), ((), ())),
                                 preferred_element_type=jnp.float32,
                                 precision=lax.Precision.HIGHEST) + bo_ref[...]
        m = jnp.max(logits, axis=1, keepdims=True)
        e = jnp.exp(logits - m)
        o_ref[...] = e / jnp.sum(e, axis=1, keepdims=True)

    return pl.pallas_call(
        body,
        out_shape=jax.ShapeDtypeStruct((g, cls), jnp.float32),
    )


def kernel(x, edge_index, i, W1, b1, W2, b2, Wo, bo):
    n, d = x.shape
    hid = W1.shape[1]
    g = 64
    cls = Wo.shape[1]
    e = edge_index.shape[1]
    assert d % 2 == 0 and hid % 2 == 0

    n_acc = NS * 8 * (-(-(n + 1) // (NS * 8)))
    epw = 2 * CH * (-(-e // (NS * 2 * CH)))
    e_pad = NS * epw

    src = edge_index[0].astype(jnp.int32)
    dst = edge_index[1].astype(jnp.int32)
    pad = e_pad - e
    if pad:
        src = jnp.concatenate([src, jnp.zeros((pad,), jnp.int32)])
        dst = jnp.concatenate([dst, jnp.full((pad,), n, jnp.int32)])
    zeros = jnp.zeros((n_acc, d // 2), jnp.float32)

    agg = _make_agg(n, n_acc, d // 2, epw)
    dense1 = _make_dense(n, d, hid)
    final = _make_final(n, hid, g, cls)

    x2 = jnp.concatenate([x[:, :d // 2], x[:, d // 2:]], axis=0)
    p = agg(x2, src, dst, zeros)
    h2s = dense1(x, p, W1, b1.reshape(1, -1))
    q = agg(h2s, src, dst, zeros)
    return final(h2s, q, W2, b2.reshape(1, -1),
                 i.astype(jnp.int32).reshape(-1, 1), Wo, bo.reshape(1, -1))
